# baseline (speedup 1.0000x reference)
import functools

import numpy as np

import jax
import jax.numpy as jnp
from jax import lax
from jax.experimental import pallas as pl
from jax.experimental.pallas import tpu as pltpu
from jax.experimental.pallas import tpu_sc as plsc

B = 64
L = 11
NPT = 2 ** L - 1
SLOTS = 2 ** L
N_PAD = B * SLOTS
D = 128
H = 128
NUM_CLASSES = 10
VOCAB = 100000


def _bitrev(p: int, bits: int) -> int:
    r = 0
    for _ in range(bits):
        r = (r << 1) | (p & 1)
        p >>= 1
    return r


def _make_node_of_slot() -> np.ndarray:
    node = np.zeros(SLOTS, np.int32)
    for l in range(L):
        n = 1 << l
        for p in range(n):
            node[n + p] = (n - 1) + _bitrev(p, l)
    return node


_NODE_OF_SLOT = _make_node_of_slot()


_NW = 32
_ROWS_PER_W = N_PAD // _NW
_CHUNK = 512
_NCHUNK = _ROWS_PER_W // _CHUNK


_G = D // 16
_CHUNK_B = 256


_HI_MASK = np.int32(-65536)


def _sc_gather_bf16(emb_i32: jax.Array, idx: jax.Array) -> jax.Array:
    nrows = idx.shape[0]
    rows_per_w = nrows // _NW
    chunk = min(_CHUNK_B, rows_per_w)
    nchunk = rows_per_w // chunk
    mesh = plsc.VectorSubcoreMesh(core_axis_name="c", subcore_axis_name="s")

    @functools.partial(
        pl.kernel,
        mesh=mesh,
        out_type=jax.ShapeDtypeStruct((nrows // 2, D), jnp.int32),
        compiler_params=pltpu.CompilerParams(use_tc_tiling_on_sc=False,
                                             needs_layout_passes=False),
        scratch_types=[
            pltpu.VMEM((chunk,), jnp.int32),
            pltpu.VMEM((chunk, D), jnp.int32),
            pltpu.VMEM((chunk, D), jnp.int32),
            pltpu.VMEM((chunk // 2, D), jnp.int32),
            pltpu.SemaphoreType.DMA,
            pltpu.SemaphoreType.DMA,
        ],
    )
    def k(emb_hbm, idx_hbm, out_hbm, idx_v, rows_a, rows_b, pack_v, sem_a, sem_b):
        info = plsc.get_sparse_core_info()
        wid = lax.axis_index("s") * info.num_cores + lax.axis_index("c")
        base = wid * rows_per_w
        bufs = (rows_a, rows_b)
        sems = (sem_a, sem_b)

        def fire(ci, buf, sem):
            start = base + ci * chunk
            pltpu.sync_copy(idx_hbm.at[pl.ds(start, chunk)], idx_v)
            return pltpu.async_copy(emb_hbm.at[idx_v], buf, sem)

        def pack_and_emit(ci, buf):
            @plsc.parallel_loop(0, chunk // 2, 1, unroll=4)
            def pair(kk):
                for m in range(_G):
                    a = buf[2 * kk, pl.ds(16 * m, 16)]
                    b = buf[2 * kk + 1, pl.ds(16 * m, 16)]
                    pack_v[kk, pl.ds(16 * m, 16)] = (
                        lax.shift_right_logical(a, 16) | (b & _HI_MASK))
            pltpu.sync_copy(
                pack_v,
                out_hbm.at[pl.ds((base + ci * chunk) // 2, chunk // 2)])

        pending = fire(0, bufs[0], sems[0])
        for ci in range(nchunk):
            cur = ci % 2
            pending.wait()
            if ci + 1 < nchunk:
                pending = fire(ci + 1, bufs[1 - cur], sems[1 - cur])
            pack_and_emit(ci, bufs[cur])

    return k(emb_i32, idx)


_T = 4
_NGROUPS = 2


def _sig(x):
    return 0.5 * jnp.tanh(0.5 * x) + 0.5


def _load_x(xv_ref, l):
    n = 1 << l
    if l == 0:
        v = pltpu.bitcast(xv_ref[:, 0:1, :], jnp.bfloat16)
        return v[:, 1:2, :].reshape(_T, D)
    v = pltpu.bitcast(xv_ref[:, n // 2:n, :], jnp.bfloat16)
    return v.reshape(_T * n, D)


def _tree_body(xv_ref, wt_ref, ut_ref, uft_ref, ufb_ref, b_ref, lint_ref,
               linb_ref, out_ref):
    f32 = jnp.float32
    xl = _load_x(xv_ref, L - 1)
    iou = jnp.dot(xl, wt_ref[:], preferred_element_type=f32) + b_ref[:]
    c = _sig(iou[:, :H]) * jnp.tanh(iou[:, 2 * H:])
    h = _sig(iou[:, H:2 * H]) * jnp.tanh(c)
    for l in range(L - 2, -1, -1):
        n = 1 << l
        f = _sig(
            jnp.dot(h, uft_ref[:], preferred_element_type=f32) + ufb_ref[:])
        fc = (f * c).reshape(_T, 2 * n, H)
        c_agg = fc[:, :n, :] + fc[:, n:, :]
        hh = h.reshape(_T, 2 * n, H)
        h_tild = (hh[:, :n, :] + hh[:, n:, :]).reshape(_T * n, H)
        xl = _load_x(xv_ref, l)
        iou = (jnp.dot(xl, wt_ref[:], preferred_element_type=f32)
               + jnp.dot(h_tild, ut_ref[:], preferred_element_type=f32)
               + b_ref[:])
        c = (_sig(iou[:, :H]) * jnp.tanh(iou[:, 2 * H:])
             + c_agg.reshape(_T * n, H))
        h = _sig(iou[:, H:2 * H]) * jnp.tanh(c)
    out_ref[0] = jnp.dot(h, lint_ref[:], preferred_element_type=f32) + linb_ref[:]


def _tree_tc(xv, wt, ut, uft, ufb, b_iou, lint, linb, *, interpret=False):
    nb = xv.shape[0]
    grid = (nb // _T,)
    full = lambda shape: pl.BlockSpec(shape, lambda g: (0,) * len(shape))
    return pl.pallas_call(
        _tree_body,
        grid=grid,
        in_specs=[
            pl.BlockSpec((_T, SLOTS // 2, D), lambda g: (g, 0, 0)),
            full((D, 3 * H)),
            full((H, 3 * H)),
            full((H, H)),
            full((1, H)),
            full((1, 3 * H)),
            full((H, 128)),
            full((1, 128)),
        ],
        out_specs=pl.BlockSpec((1, _T, 128), lambda g: (g, 0, 0)),
        out_shape=jax.ShapeDtypeStruct((nb // _T, _T, 128), jnp.float32),
        interpret=interpret,
    )(xv, wt, ut, uft, ufb, b_iou, lint, linb)


def kernel(x, emb, W_iou, U_iou, b_iou, U_f_w, U_f_b, lin_w, lin_b):
    xr = x.reshape(B, NPT)
    idx = xr[:, jnp.asarray(_NODE_OF_SLOT)].reshape(N_PAD)

    wt = W_iou.T.astype(jnp.bfloat16)
    ut = U_iou.T
    uft = U_f_w.T
    ufb = U_f_b.reshape(1, H)
    lint = jnp.pad(lin_w.T, ((0, 0), (0, 128 - NUM_CLASSES)))
    linb = jnp.pad(lin_b, (0, 128 - NUM_CLASSES)).reshape(1, 128)

    bg = B // _NGROUPS
    emb_i32 = lax.bitcast_convert_type(emb, jnp.int32)
    outs = []
    for g in range(_NGROUPS):
        idx_g = lax.dynamic_slice_in_dim(idx, g * bg * SLOTS, bg * SLOTS)
        xv_i32 = _sc_gather_bf16(emb_i32, idx_g)
        xv = xv_i32.reshape(bg, SLOTS // 2, D)
        outs.append(_tree_tc(xv, wt, ut, uft, ufb, b_iou, lint, linb))
    out = jnp.concatenate(outs, axis=0)
    return out.reshape(B, 128)[:, :NUM_CLASSES]


# speedup vs baseline: 1.2425x; 1.2425x over previous
import functools

import numpy as np

import jax
import jax.numpy as jnp
from jax import lax
from jax.experimental import pallas as pl
from jax.experimental.pallas import tpu as pltpu
from jax.experimental.pallas import tpu_sc as plsc

B = 64
L = 11
NPT = 2 ** L - 1
SLOTS = 2 ** L
N_PAD = B * SLOTS
D = 128
H = 128
NUM_CLASSES = 10
VOCAB = 100000


def _bitrev(p: int, bits: int) -> int:
    r = 0
    for _ in range(bits):
        r = (r << 1) | (p & 1)
        p >>= 1
    return r


def _make_node_of_slot() -> np.ndarray:
    node = np.zeros(SLOTS, np.int32)
    for l in range(L):
        n = 1 << l
        for p in range(n):
            node[n + p] = (n - 1) + _bitrev(p, l)
    return node


_NODE_OF_SLOT = _make_node_of_slot()


_NW = 32
_ROWS_PER_W = N_PAD // _NW
_CHUNK = 512
_NCHUNK = _ROWS_PER_W // _CHUNK


_G = D // 16
_CHUNK_B = 256


_HI_MASK = np.int32(-65536)


def _sc_gather_bf16(emb_f32: jax.Array, idx: jax.Array) -> jax.Array:
    nrows = idx.shape[0]
    rows_per_w = nrows // _NW
    chunk = min(_CHUNK_B, rows_per_w)
    nchunk = rows_per_w // chunk
    mesh = plsc.VectorSubcoreMesh(core_axis_name="c", subcore_axis_name="s")

    @functools.partial(
        pl.kernel,
        mesh=mesh,
        out_type=jax.ShapeDtypeStruct((nrows // 2, D), jnp.int32),
        compiler_params=pltpu.CompilerParams(use_tc_tiling_on_sc=False,
                                             needs_layout_passes=False),
        scratch_types=[
            pltpu.VMEM((chunk,), jnp.int32),
            pltpu.VMEM((chunk, D), jnp.float32),
            pltpu.VMEM((chunk, D), jnp.float32),
            pltpu.VMEM((chunk // 2, D), jnp.int32),
            pltpu.SemaphoreType.DMA,
            pltpu.SemaphoreType.DMA,
        ],
    )
    def k(emb_hbm, idx_hbm, out_hbm, idx_v, rows_a, rows_b, pack_v, sem_a, sem_b):
        info = plsc.get_sparse_core_info()
        wid = lax.axis_index("s") * info.num_cores + lax.axis_index("c")
        base = wid * rows_per_w
        bufs = (rows_a, rows_b)
        sems = (sem_a, sem_b)

        def fire(ci, buf, sem):
            start = base + ci * chunk
            pltpu.sync_copy(idx_hbm.at[pl.ds(start, chunk)], idx_v)
            return pltpu.async_copy(emb_hbm.at[idx_v], buf, sem)

        def pack_and_emit(ci, buf):
            @plsc.parallel_loop(0, chunk // 2, 1, unroll=4)
            def pair(kk):
                for m in range(_G):
                    a = plsc.bitcast(buf[2 * kk, pl.ds(16 * m, 16)],
                                     jnp.int32)
                    b = plsc.bitcast(buf[2 * kk + 1, pl.ds(16 * m, 16)],
                                     jnp.int32)
                    pack_v[kk, pl.ds(16 * m, 16)] = (
                        lax.shift_right_logical(a, 16) | (b & _HI_MASK))
            pltpu.sync_copy(
                pack_v,
                out_hbm.at[pl.ds((base + ci * chunk) // 2, chunk // 2)])

        pending = fire(0, bufs[0], sems[0])
        for ci in range(nchunk):
            cur = ci % 2
            pending.wait()
            if ci + 1 < nchunk:
                pending = fire(ci + 1, bufs[1 - cur], sems[1 - cur])
            pack_and_emit(ci, bufs[cur])

    return k(emb_f32, idx)


_T = 4
_NGROUPS = 4


def _sig(x):
    return 0.5 * jnp.tanh(0.5 * x) + 0.5


def _load_x(xv_ref, l):
    n = 1 << l
    if l == 0:
        v = pltpu.bitcast(xv_ref[:, 0:1, :], jnp.bfloat16)
        return v[:, 1:2, :].reshape(_T, D)
    v = pltpu.bitcast(xv_ref[:, n // 2:n, :], jnp.bfloat16)
    return v.reshape(_T * n, D)


def _tree_body(xv_ref, wt_ref, ut_ref, uft_ref, ufb_ref, b_ref, lint_ref,
               linb_ref, out_ref):
    f32 = jnp.float32
    xl = _load_x(xv_ref, L - 1)
    iou = jnp.dot(xl, wt_ref[:], preferred_element_type=f32) + b_ref[:]
    c = _sig(iou[:, :H]) * jnp.tanh(iou[:, 2 * H:])
    h = _sig(iou[:, H:2 * H]) * jnp.tanh(c)
    for l in range(L - 2, -1, -1):
        n = 1 << l
        f = _sig(
            jnp.dot(h, uft_ref[:], preferred_element_type=f32) + ufb_ref[:])
        fc = (f * c).reshape(_T, 2 * n, H)
        c_agg = fc[:, :n, :] + fc[:, n:, :]
        hh = h.reshape(_T, 2 * n, H)
        h_tild = (hh[:, :n, :] + hh[:, n:, :]).reshape(_T * n, H)
        xl = _load_x(xv_ref, l)
        iou = (jnp.dot(xl, wt_ref[:], preferred_element_type=f32)
               + jnp.dot(h_tild, ut_ref[:], preferred_element_type=f32)
               + b_ref[:])
        c = (_sig(iou[:, :H]) * jnp.tanh(iou[:, 2 * H:])
             + c_agg.reshape(_T * n, H))
        h = _sig(iou[:, H:2 * H]) * jnp.tanh(c)
    out_ref[0] = jnp.dot(h, lint_ref[:], preferred_element_type=f32) + linb_ref[:]


def _tree_tc(xv, wt, ut, uft, ufb, b_iou, lint, linb, *, interpret=False):
    nb = xv.shape[0]
    grid = (nb // _T,)
    full = lambda shape: pl.BlockSpec(shape, lambda g: (0,) * len(shape))
    return pl.pallas_call(
        _tree_body,
        grid=grid,
        in_specs=[
            pl.BlockSpec((_T, SLOTS // 2, D), lambda g: (g, 0, 0)),
            full((D, 3 * H)),
            full((H, 3 * H)),
            full((H, H)),
            full((1, H)),
            full((1, 3 * H)),
            full((H, 128)),
            full((1, 128)),
        ],
        out_specs=pl.BlockSpec((1, _T, 128), lambda g: (g, 0, 0)),
        out_shape=jax.ShapeDtypeStruct((nb // _T, _T, 128), jnp.float32),
        interpret=interpret,
    )(xv, wt, ut, uft, ufb, b_iou, lint, linb)


def kernel(x, emb, W_iou, U_iou, b_iou, U_f_w, U_f_b, lin_w, lin_b):
    xr = x.reshape(B, NPT)
    idx = xr[:, jnp.asarray(_NODE_OF_SLOT)].reshape(N_PAD)

    wt = W_iou.T.astype(jnp.bfloat16)
    ut = U_iou.T
    uft = U_f_w.T
    ufb = U_f_b.reshape(1, H)
    lint = jnp.pad(lin_w.T, ((0, 0), (0, 128 - NUM_CLASSES)))
    linb = jnp.pad(lin_b, (0, 128 - NUM_CLASSES)).reshape(1, 128)

    bg = B // _NGROUPS
    outs = []
    for g in range(_NGROUPS):
        idx_g = lax.dynamic_slice_in_dim(idx, g * bg * SLOTS, bg * SLOTS)
        xv_i32 = _sc_gather_bf16(emb, idx_g)
        xv = xv_i32.reshape(bg, SLOTS // 2, D)
        outs.append(_tree_tc(xv, wt, ut, uft, ufb, b_iou, lint, linb))
    out = jnp.concatenate(outs, axis=0)
    return out.reshape(B, 128)[:, :NUM_CLASSES]
